# baseline (speedup 1.0000x reference)
import functools

import jax
import jax.numpy as jnp
from jax import lax
from jax.experimental import pallas as pl
from jax.experimental.pallas import tpu as pltpu
from jax.experimental.pallas import tpu_sc as plsc

N = 10000
D = 128
NC = 2
NS = 16
NW = NC * NS
CHUNK = 128
LANES = 16

N_PAD = 10112
ROWS_PER_TILE = N_PAD // NS


def _mesh():
    return plsc.VectorSubcoreMesh(
        core_axis_name="c", subcore_axis_name="s", num_cores=NC, num_subcores=NS
    )


def _degree_body(dg_ch, edges_hbm, ones_hbm, zeros_hbm, hist_hbm,
                 idx_v, ones_v, acc_sh):
    c = lax.axis_index("c")
    s = lax.axis_index("s")
    row0 = s * ROWS_PER_TILE
    pltpu.sync_copy(
        zeros_hbm.at[pl.ds(row0, ROWS_PER_TILE)],
        acc_sh.at[pl.ds(row0, ROWS_PER_TILE)],
    )
    pltpu.sync_copy(ones_hbm, ones_v)
    pltpu.sync_copy(edges_hbm.at[c, s], idx_v)
    plsc.subcore_barrier()

    def chunk(j, carry):
        pltpu.sync_copy(ones_v, acc_sh.at[idx_v.at[j]], add=True)
        return carry

    lax.fori_loop(0, dg_ch, chunk, 0)
    plsc.subcore_barrier()
    pltpu.sync_copy(
        acc_sh.at[pl.ds(row0, ROWS_PER_TILE)],
        hist_hbm.at[c, pl.ds(row0, ROWS_PER_TILE)],
    )


def _degree_call(edges_dg, dg_ch):
    ones = jnp.ones((CHUNK, LANES), jnp.float32)
    zeros16 = jnp.zeros((N_PAD, LANES), jnp.float32)
    kern = pl.kernel(
        functools.partial(_degree_body, dg_ch),
        out_type=jax.ShapeDtypeStruct((NC, N_PAD, LANES), jnp.float32),
        mesh=_mesh(),
        scratch_types=[
            pltpu.VMEM((dg_ch, CHUNK), jnp.int32),
            pltpu.VMEM((CHUNK, LANES), jnp.float32),
            pltpu.VMEM_SHARED((N_PAD, LANES), jnp.float32),
        ],
        compiler_params=pltpu.CompilerParams(use_tc_tiling_on_sc=False),
    )
    return kern(edges_dg, ones, zeros16)


C0_FRAC = 53
C1_FRAC = 105


def _spmm_body(c0_ch, c1_ch, xs_hbm, edges0_hbm, edges1_hbm, zeros_hbm,
               out_hbm, src_v, dst_v, rows_v, acc_sh, sem):
    c = lax.axis_index("c")
    s = lax.axis_index("s")
    row0 = s * ROWS_PER_TILE

    @pl.when(c == 0)
    def _():
        pltpu.sync_copy(
            xs_hbm.at[pl.ds(row0, ROWS_PER_TILE)],
            acc_sh.at[pl.ds(row0, ROWS_PER_TILE)],
        )
        pltpu.sync_copy(edges0_hbm.at[0, s], src_v.at[pl.ds(0, c0_ch)])
        pltpu.sync_copy(edges0_hbm.at[1, s], dst_v.at[pl.ds(0, c0_ch)])

    @pl.when(c != 0)
    def _():
        pltpu.sync_copy(
            zeros_hbm.at[pl.ds(row0, ROWS_PER_TILE)],
            acc_sh.at[pl.ds(row0, ROWS_PER_TILE)],
        )
        pltpu.sync_copy(edges1_hbm.at[0, s], src_v.at[pl.ds(0, c1_ch)])
        pltpu.sync_copy(edges1_hbm.at[1, s], dst_v.at[pl.ds(0, c1_ch)])

    plsc.subcore_barrier()
    nch = jnp.where(c == 0, c0_ch, c1_ch)

    def chunk(j, carry):
        pltpu.async_copy(xs_hbm.at[src_v.at[j]], rows_v, sem).wait()
        pltpu.sync_copy(rows_v, acc_sh.at[dst_v.at[j]], add=True)
        return carry

    lax.fori_loop(0, nch, chunk, 0)
    plsc.subcore_barrier()
    pltpu.sync_copy(
        acc_sh.at[pl.ds(row0, ROWS_PER_TILE)],
        out_hbm.at[c, pl.ds(row0, ROWS_PER_TILE)],
    )


def _spmm_call(xs, edges0, edges1, zeros_init, c0_ch, c1_ch):
    kern = pl.kernel(
        functools.partial(_spmm_body, c0_ch, c1_ch),
        out_type=jax.ShapeDtypeStruct((NC, N_PAD, D), jnp.float32),
        mesh=_mesh(),
        scratch_types=[
            pltpu.VMEM((max(c0_ch, c1_ch), CHUNK), jnp.int32),
            pltpu.VMEM((max(c0_ch, c1_ch), CHUNK), jnp.int32),
            pltpu.VMEM((CHUNK, D), jnp.float32),
            pltpu.VMEM_SHARED((N_PAD, D), jnp.float32),
            pltpu.SemaphoreType.DMA,
        ],
    )
    return kern(xs, edges0, edges1, zeros_init)


def _scale_body(x_ref, hist_ref, xs_ref):
    h = hist_ref[...]
    norm_src = lax.rsqrt(h[0, :, 0:1] + 1.0)
    xs_ref[...] = x_ref[...] * norm_src


def _layer1_body(acc_ref, hist_ref, w_ref, b_ref, out_ref):
    h = hist_ref[...]
    norm_dst = lax.rsqrt(h[1, :, 0:1] + 1.0)
    norm_src = lax.rsqrt(h[0, :, 0:1] + 1.0)
    agg = (acc_ref[0] + acc_ref[1]) * norm_dst
    h1 = jnp.dot(agg, w_ref[...], preferred_element_type=jnp.float32) + b_ref[...]
    out_ref[...] = jnp.maximum(h1, 0.0) * norm_src


def _layer2_body(acc_ref, hist_ref, w_ref, b_ref, out_ref):
    h = hist_ref[...]
    norm_dst = lax.rsqrt(h[1, :, 0:1] + 1.0)
    agg = (acc_ref[0] + acc_ref[1]) * norm_dst
    out_ref[...] = (
        jnp.dot(agg, w_ref[...], preferred_element_type=jnp.float32) + b_ref[...]
    )


def _tc_call(body, out_shape, *args):
    return pl.pallas_call(
        body, out_shape=jax.ShapeDtypeStruct(out_shape, jnp.float32)
    )(*args)


def kernel(features, edge_index, W1, b1, W2, b2):
    n, d = features.shape
    e = edge_index.shape[1]
    e_pad = -(-e // (NS * CHUNK)) * (NS * CHUNK)
    dg_ch = e_pad // (NS * CHUNK)
    tot_ch = dg_ch
    c0_ch = max(1, (tot_ch * C0_FRAC) // (C0_FRAC + C1_FRAC))
    c1_ch = tot_ch - c0_ch

    ei = jnp.pad(edge_index, ((0, 0), (0, e_pad - e)), constant_values=N)
    edges_dg = ei.reshape(2, NS, dg_ch, CHUNK)
    n0 = NS * c0_ch * CHUNK
    edges0 = ei[:, :n0].reshape(2, NS, c0_ch, CHUNK)
    edges1 = ei[:, n0:].reshape(2, NS, c1_ch, CHUNK)
    x_pad = jnp.pad(features, ((0, N_PAD - n), (0, 0)))
    zeros_init = jnp.zeros((N_PAD, D), jnp.float32)

    hist = _degree_call(edges_dg, dg_ch)
    xs1 = _tc_call(_scale_body, (N_PAD, D), x_pad, hist)
    acc1 = _spmm_call(xs1, edges0, edges1, zeros_init, c0_ch, c1_ch)
    xs2 = _tc_call(_layer1_body, (N_PAD, D), acc1, hist, W1, b1.reshape(1, D))
    acc2 = _spmm_call(xs2, edges0, edges1, zeros_init, c0_ch, c1_ch)
    out = _tc_call(_layer2_body, (N_PAD, D), acc2, hist, W2, b2.reshape(1, D))
    return out[:n]


# speedup vs baseline: 1.1069x; 1.1069x over previous
import functools

import jax
import jax.numpy as jnp
from jax import lax
from jax.experimental import pallas as pl
from jax.experimental.pallas import tpu as pltpu
from jax.experimental.pallas import tpu_sc as plsc

N = 10000
D = 128
NC = 2
NS = 16
NW = NC * NS
CHUNK = 128
LANES = 16

N_PAD = 10112
ROWS_PER_TILE = N_PAD // NS


def _mesh():
    return plsc.VectorSubcoreMesh(
        core_axis_name="c", subcore_axis_name="s", num_cores=NC, num_subcores=NS
    )


def _degree_body(dg_ch, edges_hbm, ones_hbm, zeros_hbm, hist_hbm,
                 idx_v, ones_v, acc_sh):
    c = lax.axis_index("c")
    s = lax.axis_index("s")
    row0 = s * ROWS_PER_TILE
    pltpu.sync_copy(
        zeros_hbm.at[pl.ds(row0, ROWS_PER_TILE)],
        acc_sh.at[pl.ds(row0, ROWS_PER_TILE)],
    )
    pltpu.sync_copy(ones_hbm, ones_v)
    pltpu.sync_copy(edges_hbm.at[c, s], idx_v)
    plsc.subcore_barrier()

    def chunk(j, carry):
        pltpu.sync_copy(ones_v, acc_sh.at[idx_v.at[j]], add=True)
        return carry

    lax.fori_loop(0, dg_ch, chunk, 0)
    plsc.subcore_barrier()
    pltpu.sync_copy(
        acc_sh.at[pl.ds(row0, ROWS_PER_TILE)],
        hist_hbm.at[c, pl.ds(row0, ROWS_PER_TILE)],
    )


def _degree_call(edges_dg, dg_ch):
    ones = jnp.ones((CHUNK, LANES), jnp.float32)
    zeros16 = jnp.zeros((N_PAD, LANES), jnp.float32)
    kern = pl.kernel(
        functools.partial(_degree_body, dg_ch),
        out_type=jax.ShapeDtypeStruct((NC, N_PAD, LANES), jnp.float32),
        mesh=_mesh(),
        scratch_types=[
            pltpu.VMEM((dg_ch, CHUNK), jnp.int32),
            pltpu.VMEM((CHUNK, LANES), jnp.float32),
            pltpu.VMEM_SHARED((N_PAD, LANES), jnp.float32),
        ],
        compiler_params=pltpu.CompilerParams(use_tc_tiling_on_sc=False),
    )
    return kern(edges_dg, ones, zeros16)


C0_FRAC = 57
C1_FRAC = 101


def _spmm_body(c0_ch, c1_ch, xs_hbm, edges0_hbm, edges1_hbm, zeros_hbm,
               out_hbm, src_v, dst_v, rows_v, acc_sh, sem):
    c = lax.axis_index("c")
    s = lax.axis_index("s")
    row0 = s * ROWS_PER_TILE

    @pl.when(c == 0)
    def _():
        pltpu.sync_copy(
            xs_hbm.at[pl.ds(row0, ROWS_PER_TILE)],
            acc_sh.at[pl.ds(row0, ROWS_PER_TILE)],
        )
        pltpu.sync_copy(edges0_hbm.at[0, s], src_v.at[pl.ds(0, c0_ch)])
        pltpu.sync_copy(edges0_hbm.at[1, s], dst_v.at[pl.ds(0, c0_ch)])

    @pl.when(c != 0)
    def _():
        pltpu.sync_copy(
            zeros_hbm.at[pl.ds(row0, ROWS_PER_TILE)],
            acc_sh.at[pl.ds(row0, ROWS_PER_TILE)],
        )
        pltpu.sync_copy(edges1_hbm.at[0, s], src_v.at[pl.ds(0, c1_ch)])
        pltpu.sync_copy(edges1_hbm.at[1, s], dst_v.at[pl.ds(0, c1_ch)])

    plsc.subcore_barrier()
    nch = jnp.where(c == 0, c0_ch, c1_ch)

    def chunk(j, carry):
        pltpu.async_copy(xs_hbm.at[src_v.at[j]], rows_v, sem).wait()
        pltpu.sync_copy(rows_v, acc_sh.at[dst_v.at[j]], add=True)
        return carry

    lax.fori_loop(0, nch, chunk, 0)
    plsc.subcore_barrier()
    pltpu.sync_copy(
        acc_sh.at[pl.ds(row0, ROWS_PER_TILE)],
        out_hbm.at[c, pl.ds(row0, ROWS_PER_TILE)],
    )


def _spmm_call(xs, edges0, edges1, zeros_init, c0_ch, c1_ch):
    kern = pl.kernel(
        functools.partial(_spmm_body, c0_ch, c1_ch),
        out_type=jax.ShapeDtypeStruct((NC, N_PAD, D), jnp.float32),
        mesh=_mesh(),
        scratch_types=[
            pltpu.VMEM((max(c0_ch, c1_ch), CHUNK), jnp.int32),
            pltpu.VMEM((max(c0_ch, c1_ch), CHUNK), jnp.int32),
            pltpu.VMEM((CHUNK, D), jnp.float32),
            pltpu.VMEM_SHARED((N_PAD, D), jnp.float32),
            pltpu.SemaphoreType.DMA,
        ],
    )
    return kern(xs, edges0, edges1, zeros_init)


def _scale_body(x_ref, hist_ref, xs_ref):
    h = hist_ref[...]
    norm_src = lax.rsqrt(h[0, :, 0:1] + 1.0)
    xs_ref[...] = x_ref[...] * norm_src


def _layer1_body(acc_ref, hist_ref, w_ref, b_ref, out_ref):
    h = hist_ref[...]
    norm_dst = lax.rsqrt(h[1, :, 0:1] + 1.0)
    norm_src = lax.rsqrt(h[0, :, 0:1] + 1.0)
    agg = (acc_ref[0] + acc_ref[1]) * norm_dst
    h1 = jnp.dot(agg, w_ref[...], preferred_element_type=jnp.float32) + b_ref[...]
    out_ref[...] = jnp.maximum(h1, 0.0) * norm_src


def _layer2_body(acc_ref, hist_ref, w_ref, b_ref, out_ref):
    h = hist_ref[...]
    norm_dst = lax.rsqrt(h[1, :, 0:1] + 1.0)
    agg = (acc_ref[0] + acc_ref[1]) * norm_dst
    out_ref[...] = (
        jnp.dot(agg, w_ref[...], preferred_element_type=jnp.float32) + b_ref[...]
    )


def _tc_call(body, out_shape, *args):
    return pl.pallas_call(
        body, out_shape=jax.ShapeDtypeStruct(out_shape, jnp.float32)
    )(*args)


def kernel(features, edge_index, W1, b1, W2, b2):
    n, d = features.shape
    e = edge_index.shape[1]
    e_pad = -(-e // (NS * CHUNK)) * (NS * CHUNK)
    dg_ch = e_pad // (NS * CHUNK)
    tot_ch = dg_ch
    c0_ch = max(1, (tot_ch * C0_FRAC) // (C0_FRAC + C1_FRAC))
    c1_ch = tot_ch - c0_ch

    ei = jnp.pad(edge_index, ((0, 0), (0, e_pad - e)), constant_values=N)
    edges_dg = ei.reshape(2, NS, dg_ch, CHUNK)
    n0 = NS * c0_ch * CHUNK
    edges0 = ei[:, :n0].reshape(2, NS, c0_ch, CHUNK)
    edges1 = ei[:, n0:].reshape(2, NS, c1_ch, CHUNK)
    x_pad = jnp.pad(features, ((0, N_PAD - n), (0, 0)))
    zeros_init = jnp.zeros((N_PAD, D), jnp.float32)

    hist = _degree_call(edges_dg, dg_ch)
    xs1 = _tc_call(_scale_body, (N_PAD, D), x_pad, hist)
    acc1 = _spmm_call(xs1, edges0, edges1, zeros_init, c0_ch, c1_ch)
    xs2 = _tc_call(_layer1_body, (N_PAD, D), acc1, hist, W1, b1.reshape(1, D))
    acc2 = _spmm_call(xs2, edges0, edges1, zeros_init, c0_ch, c1_ch)
    out = _tc_call(_layer2_body, (N_PAD, D), acc2, hist, W2, b2.reshape(1, D))
    return out[:n]
